# Initial kernel scaffold; baseline (speedup 1.0000x reference)
import jax
import jax.numpy as jnp
from jax.experimental import pallas as pl


def kernel(x, edge_index, edge_attr, params):
    raise NotImplementedError("write your pallas kernel here")


# speedup vs baseline: 2.4252x; 2.4252x over previous
import functools

import jax
import jax.numpy as jnp
from jax import lax
from jax.experimental import pallas as pl
from jax.experimental.pallas import tpu as pltpu
from jax.experimental.pallas import tpu_sc as plsc

EPS = 1e-5
NC, NS = 2, 16
NW = NC * NS
CHUNK = 128
BE = 2048
BN = 1000

_SC_PARAMS = pltpu.CompilerParams(use_tc_tiling_on_sc=False)


def _mk_mesh():
    return plsc.VectorSubcoreMesh(core_axis_name="c", subcore_axis_name="s",
                                  num_cores=NC, num_subcores=NS)


def _wid():
    return lax.axis_index("s") * NC + lax.axis_index("c")


def _ln_act(h, g, b):
    m = jnp.mean(h, axis=-1, keepdims=True)
    d = h - m
    v = jnp.mean(d * d, axis=-1, keepdims=True)
    return jnp.maximum(d * lax.rsqrt(v + EPS) * g + b, 0.0)


def _dot(a, w):
    return jnp.dot(a, w, preferred_element_type=jnp.float32)


def _enc_body(ea, w0, b0, g0, bb0, w1, b1, g1, bb1, w2, b2, out):
    h = _ln_act(_dot(ea[...], w0[...]) + b0[...], g0[...], bb0[...])
    h = _ln_act(_dot(h, w1[...]) + b1[...], g1[...], bb1[...])
    out[...] = _dot(h, w2[...]) + b2[...]


def _pre_body(x, wcat, bcat, t1, t2):
    r = _dot(x[...], wcat[...]) + bcat[...]
    t1[...] = r[:, :32]
    t2[...] = r[:, 32:]


def _edgeflow_body(s1, s2, e, wc, ge1, bbe1, we2, be2, ge2, bbe2,
                   vb, gf1, bbf1, wf2, bf2, gf2, bbf2, e_out, f_out):
    h1 = _ln_act(s1[...] + s2[...][:, :32] + _dot(e[...], wc[...]),
                 ge1[...], bbe1[...])
    e_new = _ln_act(_dot(h1, we2[...]) + be2[...], ge2[...], bbe2[...])
    g1 = _ln_act(s2[...][:, 32:] + _dot(e_new, vb[...]), gf1[...], bbf1[...])
    f_out[...] = _ln_act(_dot(g1, wf2[...]) + bf2[...], gf2[...], bbf2[...])
    e_out[...] = e_new


def _node_body(p, wn, bn, gn, bbn, out):
    a = p[0] + p[1]
    out[...] = _ln_act(_dot(a, wn[...]) + bn[...], gn[...], bbn[...])


def _full(shape):
    return pl.BlockSpec(shape, lambda i: tuple(0 for _ in shape))


def _rows(bs, width):
    return pl.BlockSpec((bs, width), lambda i: (i, 0))


def _make_sc_kernels(n_nodes, n_acc, cpw):
    r2 = NW * cpw
    e_pad = r2 * CHUNK
    rpt = n_acc // NS

    @functools.partial(
        pl.kernel, mesh=_mk_mesh(), compiler_params=_SC_PARAMS,
        out_type=jax.ShapeDtypeStruct((r2, CHUNK), jnp.int32),
        scratch_types=[
            pltpu.VMEM((cpw, CHUNK), jnp.int32),
            pltpu.VMEM((cpw, CHUNK), jnp.int32),
        ],
    )
    def prep_k(row_hbm, col_hbm, out_hbm, rowv, colv):
        w = _wid()
        pltpu.sync_copy(row_hbm.at[pl.ds(w * cpw, cpw)], rowv)
        pltpu.sync_copy(col_hbm.at[pl.ds(w * cpw, cpw)], colv)

        def body(j, carry):
            for k in range(CHUNK // 16):
                r = rowv[j, pl.ds(k * 16, 16)]
                c = colv[j, pl.ds(k * 16, 16)]
                rowv[j, pl.ds(k * 16, 16)] = jnp.where(r == c, n_nodes, r)
            return carry

        lax.fori_loop(0, cpw, body, 0)
        pltpu.sync_copy(rowv, out_hbm.at[pl.ds(w * cpw, cpw)])

    @functools.partial(
        pl.kernel, mesh=_mk_mesh(), compiler_params=_SC_PARAMS,
        out_type=(jax.ShapeDtypeStruct((e_pad, 32), jnp.float32),
                  jax.ShapeDtypeStruct((e_pad, 64), jnp.float32)),
        scratch_types=[
            pltpu.VMEM((cpw, CHUNK), jnp.int32),
            pltpu.VMEM((cpw, CHUNK), jnp.int32),
            pltpu.VMEM((CHUNK, 32), jnp.float32),
            pltpu.VMEM((CHUNK, 64), jnp.float32),
            pltpu.SemaphoreType.DMA,
            pltpu.SemaphoreType.DMA,
        ],
    )
    def gather_k(t1_hbm, t2_hbm, row_hbm, col_hbm, s1_hbm, s2_hbm,
                 rowv, colv, buf1, buf2, sem1, sem2):
        w = _wid()
        pltpu.sync_copy(row_hbm.at[pl.ds(w * cpw, cpw)], rowv)
        pltpu.sync_copy(col_hbm.at[pl.ds(w * cpw, cpw)], colv)

        def body(j, carry):
            d1 = pltpu.async_copy(t1_hbm.at[rowv.at[j]], buf1, sem1)
            d2 = pltpu.async_copy(t2_hbm.at[colv.at[j]], buf2, sem2)
            d1.wait()
            d2.wait()
            off = (w * cpw + j) * CHUNK
            pltpu.sync_copy(buf1, s1_hbm.at[pl.ds(off, CHUNK)])
            pltpu.sync_copy(buf2, s2_hbm.at[pl.ds(off, CHUNK)])
            return carry

        lax.fori_loop(0, cpw, body, 0)

    @functools.partial(
        pl.kernel, mesh=_mk_mesh(), compiler_params=_SC_PARAMS,
        out_type=jax.ShapeDtypeStruct((NC, n_acc, 128), jnp.float32),
        scratch_types=[
            pltpu.VMEM((cpw, CHUNK), jnp.int32),
            pltpu.VMEM((CHUNK, 128), jnp.float32),
            pltpu.VMEM((16, 128), jnp.float32),
            pltpu.VMEM_SHARED((n_acc, 128), jnp.float32),
        ],
    )
    def scatter_k(fm_hbm, idx_hbm, out_hbm, idxv, buf, zbuf, acc):
        cid = lax.axis_index("c")
        sid = lax.axis_index("s")
        w = _wid()
        for r in range(16):
            for k in range(8):
                zbuf[r, pl.ds(k * 16, 16)] = jnp.zeros((16,), jnp.float32)
        base = sid * rpt

        def zbody(k, carry):
            pltpu.sync_copy(zbuf, acc.at[pl.ds(base + k * 16, 16)])
            return carry

        lax.fori_loop(0, rpt // 16, zbody, 0)
        pltpu.sync_copy(idx_hbm.at[pl.ds(w * cpw, cpw)], idxv)
        plsc.subcore_barrier()

        def body(j, carry):
            pltpu.sync_copy(fm_hbm.at[pl.ds((w * cpw + j) * CHUNK, CHUNK)], buf)
            pltpu.sync_copy(buf, acc.at[idxv.at[j]], add=True)
            return carry

        lax.fori_loop(0, cpw, body, 0)
        plsc.subcore_barrier()
        pltpu.sync_copy(acc.at[pl.ds(base, rpt)],
                        out_hbm.at[cid].at[pl.ds(base, rpt)])

    return prep_k, gather_k, scatter_k


def kernel(x, edge_index, edge_attr, params):
    n_nodes, d_node = x.shape
    n_edges = edge_attr.shape[0]
    f32 = jnp.float32

    cpw = -(-n_edges // (NW * CHUNK))
    cpw = ((cpw + 7) // 8) * 8
    e_pad = NW * cpw * CHUNK
    r2 = NW * cpw
    rpt = 16 * (-(-(-(-(n_nodes + 1) // NS)) // 16))
    n_acc = rpt * NS

    prep_k, gather_k, scatter_k = _make_sc_kernels(n_nodes, n_acc, cpw)

    row = edge_index[0]
    col = edge_index[1]
    pad = e_pad - n_edges
    rowp = jnp.concatenate([row, jnp.zeros((pad,), jnp.int32)]).reshape(r2, CHUNK)
    colp = jnp.concatenate([col, jnp.zeros((pad,), jnp.int32)]).reshape(r2, CHUNK)
    eap = jnp.concatenate([edge_attr, jnp.zeros((pad, edge_attr.shape[1]), f32)])

    (we0, be0, ge0, bbe0), (we1_, be1_, ge1_, bbe1_), (wef, bef) = params['enc']
    (we1, be1, ge1, bbe1), (we2, be2, ge2, bbe2) = params['edge']
    (wf1, bf1, gf1, bbf1), (wf2, bf2, gf2, bbf2) = params['flow']
    ((wn, bn, gn, bbn),) = params['node']

    def r2d(v):
        return v.reshape(1, -1)

    wa = we1[:d_node]
    wb = we1[d_node:2 * d_node]
    wc = we1[2 * d_node:]
    va = wf1[:d_node]
    vb = wf1[d_node:]
    wcat = jnp.concatenate([wa, wb, va], axis=1)
    bcat = jnp.concatenate([be1, jnp.zeros((32,), f32), bf1]).reshape(1, 96)

    ge = e_pad // BE
    gn_blocks = n_nodes // BN

    e0 = pl.pallas_call(
        _enc_body,
        grid=(ge,),
        in_specs=[_rows(BE, 16),
                  _full((16, 32)), _full((1, 32)), _full((1, 32)), _full((1, 32)),
                  _full((32, 16)), _full((1, 16)), _full((1, 16)), _full((1, 16)),
                  _full((16, 16)), _full((1, 16))],
        out_specs=_rows(BE, 16),
        out_shape=jax.ShapeDtypeStruct((e_pad, 16), f32),
    )(eap, we0, r2d(be0), r2d(ge0), r2d(bbe0),
      we1_, r2d(be1_), r2d(ge1_), r2d(bbe1_), wef, r2d(bef))

    rowe = prep_k(rowp, colp)

    edgeflow = pl.pallas_call(
        _edgeflow_body,
        grid=(ge,),
        in_specs=[_rows(BE, 32), _rows(BE, 64), _rows(BE, 16),
                  _full((16, 32)), _full((1, 32)), _full((1, 32)),
                  _full((32, 16)), _full((1, 16)), _full((1, 16)), _full((1, 16)),
                  _full((16, 32)), _full((1, 32)), _full((1, 32)),
                  _full((32, 128)), _full((1, 128)), _full((1, 128)), _full((1, 128))],
        out_specs=(_rows(BE, 16), _rows(BE, 128)),
        out_shape=(jax.ShapeDtypeStruct((e_pad, 16), f32),
                   jax.ShapeDtypeStruct((e_pad, 128), f32)),
    )

    precompute = pl.pallas_call(
        _pre_body,
        grid=(gn_blocks,),
        in_specs=[_rows(BN, d_node), _full((d_node, 96)), _full((1, 96))],
        out_specs=(_rows(BN, 32), _rows(BN, 64)),
        out_shape=(jax.ShapeDtypeStruct((n_nodes, 32), f32),
                   jax.ShapeDtypeStruct((n_nodes, 64), f32)),
    )

    node_mlp = pl.pallas_call(
        _node_body,
        grid=(gn_blocks,),
        in_specs=[pl.BlockSpec((NC, BN, d_node), lambda i: (0, i, 0)),
                  _full((d_node, d_node)), _full((1, d_node)),
                  _full((1, d_node)), _full((1, d_node))],
        out_specs=_rows(BN, d_node),
        out_shape=jax.ShapeDtypeStruct((n_nodes, d_node), f32),
    )

    e = e0
    for _ in range(4):
        t1, t2 = precompute(x, wcat, bcat)
        s1, s2 = gather_k(t1, t2, rowp, colp)
        e, fm = edgeflow(s1, s2, e,
                         wc, r2d(ge1), r2d(bbe1),
                         we2, r2d(be2), r2d(ge2), r2d(bbe2),
                         vb, r2d(gf1), r2d(bbf1),
                         wf2, r2d(bf2), r2d(gf2), r2d(bbf2))
        partials = scatter_k(fm, rowe)
        x = node_mlp(partials, wn, r2d(bn), r2d(gn), r2d(bbn))

    return x, e[:n_edges]


# speedup vs baseline: 2.6370x; 1.0873x over previous
import functools

import jax
import jax.numpy as jnp
from jax import lax
from jax.experimental import pallas as pl
from jax.experimental.pallas import tpu as pltpu
from jax.experimental.pallas import tpu_sc as plsc

EPS = 1e-5
NC, NS = 2, 16
NW = NC * NS
CHUNK = 128
BE = 2048
BN = 1000

_SC_PARAMS = pltpu.CompilerParams(use_tc_tiling_on_sc=False)


def _mk_mesh():
    return plsc.VectorSubcoreMesh(core_axis_name="c", subcore_axis_name="s",
                                  num_cores=NC, num_subcores=NS)


def _wid():
    return lax.axis_index("s") * NC + lax.axis_index("c")


def _ln_act(h, g, b):
    m = jnp.mean(h, axis=-1, keepdims=True)
    d = h - m
    v = jnp.mean(d * d, axis=-1, keepdims=True)
    return jnp.maximum(d * lax.rsqrt(v + EPS) * g + b, 0.0)


def _dot(a, w):
    return jnp.dot(a, w, preferred_element_type=jnp.float32)


def _enc_body(ea, w0, b0, g0, bb0, w1, b1, g1, bb1, w2, b2, out):
    h = _ln_act(_dot(ea[...], w0[...]) + b0[...], g0[...], bb0[...])
    h = _ln_act(_dot(h, w1[...]) + b1[...], g1[...], bb1[...])
    out[...] = _dot(h, w2[...]) + b2[...]


def _pre_body(x, wcat, bcat, t1, t2):
    r = _dot(x[...], wcat[...]) + bcat[...]
    t1[...] = r[:, :32]
    t2[...] = r[:, 32:]


def _edgeflow_body(s1, s2, e, wc, ge1, bbe1, we2, be2, ge2, bbe2,
                   vb, gf1, bbf1, wf2, bf2, gf2, bbf2, e_out, f_out):
    h1 = _ln_act(s1[...] + s2[...][:, :32] + _dot(e[...], wc[...]),
                 ge1[...], bbe1[...])
    e_new = _ln_act(_dot(h1, we2[...]) + be2[...], ge2[...], bbe2[...])
    g1 = _ln_act(s2[...][:, 32:] + _dot(e_new, vb[...]), gf1[...], bbf1[...])
    f_out[...] = _ln_act(_dot(g1, wf2[...]) + bf2[...], gf2[...], bbf2[...])
    e_out[...] = e_new


def _node_body(p, wn, bn, gn, bbn, out):
    a = p[0] + p[1]
    out[...] = _ln_act(_dot(a, wn[...]) + bn[...], gn[...], bbn[...])


def _full(shape):
    return pl.BlockSpec(shape, lambda i: tuple(0 for _ in shape))


def _rows(bs, width):
    return pl.BlockSpec((bs, width), lambda i: (i, 0))


def _make_sc_kernels(n_nodes, n_acc, cpw):
    r2 = NW * cpw
    e_pad = r2 * CHUNK
    rpt = n_acc // NS

    @functools.partial(
        pl.kernel, mesh=_mk_mesh(), compiler_params=_SC_PARAMS,
        out_type=jax.ShapeDtypeStruct((r2, CHUNK), jnp.int32),
        scratch_types=[
            pltpu.VMEM((cpw, CHUNK), jnp.int32),
            pltpu.VMEM((cpw, CHUNK), jnp.int32),
        ],
    )
    def prep_k(row_hbm, col_hbm, out_hbm, rowv, colv):
        w = _wid()
        pltpu.sync_copy(row_hbm.at[pl.ds(w * cpw, cpw)], rowv)
        pltpu.sync_copy(col_hbm.at[pl.ds(w * cpw, cpw)], colv)

        def body(j, carry):
            for k in range(CHUNK // 16):
                r = rowv[j, pl.ds(k * 16, 16)]
                c = colv[j, pl.ds(k * 16, 16)]
                rowv[j, pl.ds(k * 16, 16)] = jnp.where(r == c, n_nodes, r)
            return carry

        lax.fori_loop(0, cpw, body, 0)
        pltpu.sync_copy(rowv, out_hbm.at[pl.ds(w * cpw, cpw)])

    kg = 4
    ngg = cpw // kg

    @functools.partial(
        pl.kernel, mesh=_mk_mesh(), compiler_params=_SC_PARAMS,
        out_type=(jax.ShapeDtypeStruct((e_pad, 32), jnp.float32),
                  jax.ShapeDtypeStruct((e_pad, 64), jnp.float32)),
        scratch_types=[
            pltpu.VMEM((cpw, CHUNK), jnp.int32),
            pltpu.VMEM((cpw, CHUNK), jnp.int32),
            pltpu.VMEM((2, kg * CHUNK, 32), jnp.float32),
            pltpu.VMEM((2, kg * CHUNK, 64), jnp.float32),
            pltpu.SemaphoreType.DMA,
            pltpu.SemaphoreType.DMA,
            pltpu.SemaphoreType.DMA,
            pltpu.SemaphoreType.DMA,
        ],
    )
    def gather_k(t1_hbm, t2_hbm, row_hbm, col_hbm, s1_hbm, s2_hbm,
                 rowv, colv, buf1, buf2, gsem_a, gsem_b, ssem_a, ssem_b):
        w = _wid()
        pltpu.sync_copy(row_hbm.at[pl.ds(w * cpw, cpw)], rowv)
        pltpu.sync_copy(col_hbm.at[pl.ds(w * cpw, cpw)], colv)

        def gathers(g, s, sem, start):
            for i in range(kg):
                for tab, idx, buf in ((t1_hbm, rowv, buf1),
                                      (t2_hbm, colv, buf2)):
                    d = pltpu.make_async_copy(
                        tab.at[idx.at[g * kg + i]],
                        buf.at[s].at[pl.ds(i * CHUNK, CHUNK)], sem)
                    d.start() if start else d.wait()

        def stores(g, s, sem, start):
            off = (w * cpw + g * kg) * CHUNK
            for out, buf in ((s1_hbm, buf1), (s2_hbm, buf2)):
                d = pltpu.make_async_copy(
                    buf.at[s], out.at[pl.ds(off, kg * CHUNK)], sem)
                d.start() if start else d.wait()

        def process(g, s, gsem, ssem, gsem_n, ssem_n):
            @pl.when(g >= 1)
            def _():
                stores(g - 1, 1 - s, ssem_n, False)

            @pl.when(g + 1 < ngg)
            def _():
                gathers(g + 1, 1 - s, gsem_n, True)

            gathers(g, s, gsem, False)
            stores(g, s, ssem, True)

        gathers(0, 0, gsem_a, True)

        def body(g2, carry):
            process(2 * g2, 0, gsem_a, ssem_a, gsem_b, ssem_b)
            process(2 * g2 + 1, 1, gsem_b, ssem_b, gsem_a, ssem_a)
            return carry

        lax.fori_loop(0, ngg // 2, body, 0)
        stores(ngg - 1, 1, ssem_b, False)

    ks = 1
    ngs = cpw // ks

    @functools.partial(
        pl.kernel, mesh=_mk_mesh(), compiler_params=_SC_PARAMS,
        out_type=jax.ShapeDtypeStruct((NC, n_acc, 128), jnp.float32),
        scratch_types=[
            pltpu.VMEM((cpw, CHUNK), jnp.int32),
            pltpu.VMEM((2, ks * CHUNK, 128), jnp.float32),
            pltpu.VMEM((16, 128), jnp.float32),
            pltpu.VMEM_SHARED((n_acc, 128), jnp.float32),
            pltpu.SemaphoreType.DMA,
            pltpu.SemaphoreType.DMA,
            pltpu.SemaphoreType.DMA,
            pltpu.SemaphoreType.DMA,
            pltpu.SemaphoreType.DMA,
        ],
    )
    def scatter_k(fm_hbm, idx_hbm, out_hbm, idxv, buf, zbuf, acc,
                  lsem_a, lsem_b, asem_a, asem_b, zsem):
        cid = lax.axis_index("c")
        sid = lax.axis_index("s")
        w = _wid()
        for r in range(16):
            for k in range(8):
                zbuf[r, pl.ds(k * 16, 16)] = jnp.zeros((16,), jnp.float32)
        base = sid * rpt

        def zbody(k, carry):
            pltpu.async_copy(zbuf, acc.at[pl.ds(base + k * 16, 16)], zsem)
            return carry

        lax.fori_loop(0, rpt // 16, zbody, 0)
        pltpu.sync_copy(idx_hbm.at[pl.ds(w * cpw, cpw)], idxv)

        def zdrain(k, carry):
            pltpu.make_async_copy(
                zbuf, acc.at[pl.ds(base + k * 16, 16)], zsem).wait()
            return carry

        lax.fori_loop(0, rpt // 16, zdrain, 0)
        plsc.subcore_barrier()

        def load(g, s, sem, start):
            d = pltpu.make_async_copy(
                fm_hbm.at[pl.ds((w * cpw + g * ks) * CHUNK, ks * CHUNK)],
                buf.at[s], sem)
            d.start() if start else d.wait()

        def adds(g, s, sem, start):
            for i in range(ks):
                d = pltpu.make_async_copy(
                    buf.at[s].at[pl.ds(i * CHUNK, CHUNK)],
                    acc.at[idxv.at[g * ks + i]], sem)
                if start:
                    pltpu.async_copy(
                        buf.at[s].at[pl.ds(i * CHUNK, CHUNK)],
                        acc.at[idxv.at[g * ks + i]], sem, add=True)
                else:
                    d.wait()

        def process(g, s, lsem, asem, lsem_n, asem_n):
            @pl.when(g >= 1)
            def _():
                adds(g - 1, 1 - s, asem_n, False)

            @pl.when(g + 1 < ngs)
            def _():
                load(g + 1, 1 - s, lsem_n, True)

            load(g, s, lsem, False)
            adds(g, s, asem, True)

        load(0, 0, lsem_a, True)

        def body(g2, carry):
            process(2 * g2, 0, lsem_a, asem_a, lsem_b, asem_b)
            process(2 * g2 + 1, 1, lsem_b, asem_b, lsem_a, asem_a)
            return carry

        lax.fori_loop(0, ngs // 2, body, 0)
        adds(ngs - 1, 1, asem_b, False)
        plsc.subcore_barrier()
        pltpu.sync_copy(acc.at[pl.ds(base, rpt)],
                        out_hbm.at[cid].at[pl.ds(base, rpt)])

    return prep_k, gather_k, scatter_k


def kernel(x, edge_index, edge_attr, params):
    n_nodes, d_node = x.shape
    n_edges = edge_attr.shape[0]
    f32 = jnp.float32

    cpw = -(-n_edges // (NW * CHUNK))
    cpw = ((cpw + 7) // 8) * 8
    e_pad = NW * cpw * CHUNK
    r2 = NW * cpw
    rpt = 16 * (-(-(-(-(n_nodes + 1) // NS)) // 16))
    n_acc = rpt * NS

    prep_k, gather_k, scatter_k = _make_sc_kernels(n_nodes, n_acc, cpw)

    row = edge_index[0]
    col = edge_index[1]
    pad = e_pad - n_edges
    rowp = jnp.concatenate([row, jnp.zeros((pad,), jnp.int32)]).reshape(r2, CHUNK)
    colp = jnp.concatenate([col, jnp.zeros((pad,), jnp.int32)]).reshape(r2, CHUNK)
    eap = jnp.concatenate([edge_attr, jnp.zeros((pad, edge_attr.shape[1]), f32)])

    (we0, be0, ge0, bbe0), (we1_, be1_, ge1_, bbe1_), (wef, bef) = params['enc']
    (we1, be1, ge1, bbe1), (we2, be2, ge2, bbe2) = params['edge']
    (wf1, bf1, gf1, bbf1), (wf2, bf2, gf2, bbf2) = params['flow']
    ((wn, bn, gn, bbn),) = params['node']

    def r2d(v):
        return v.reshape(1, -1)

    wa = we1[:d_node]
    wb = we1[d_node:2 * d_node]
    wc = we1[2 * d_node:]
    va = wf1[:d_node]
    vb = wf1[d_node:]
    wcat = jnp.concatenate([wa, wb, va], axis=1)
    bcat = jnp.concatenate([be1, jnp.zeros((32,), f32), bf1]).reshape(1, 96)

    ge = e_pad // BE
    gn_blocks = n_nodes // BN

    e0 = pl.pallas_call(
        _enc_body,
        grid=(ge,),
        in_specs=[_rows(BE, 16),
                  _full((16, 32)), _full((1, 32)), _full((1, 32)), _full((1, 32)),
                  _full((32, 16)), _full((1, 16)), _full((1, 16)), _full((1, 16)),
                  _full((16, 16)), _full((1, 16))],
        out_specs=_rows(BE, 16),
        out_shape=jax.ShapeDtypeStruct((e_pad, 16), f32),
    )(eap, we0, r2d(be0), r2d(ge0), r2d(bbe0),
      we1_, r2d(be1_), r2d(ge1_), r2d(bbe1_), wef, r2d(bef))

    rowe = prep_k(rowp, colp)

    edgeflow = pl.pallas_call(
        _edgeflow_body,
        grid=(ge,),
        in_specs=[_rows(BE, 32), _rows(BE, 64), _rows(BE, 16),
                  _full((16, 32)), _full((1, 32)), _full((1, 32)),
                  _full((32, 16)), _full((1, 16)), _full((1, 16)), _full((1, 16)),
                  _full((16, 32)), _full((1, 32)), _full((1, 32)),
                  _full((32, 128)), _full((1, 128)), _full((1, 128)), _full((1, 128))],
        out_specs=(_rows(BE, 16), _rows(BE, 128)),
        out_shape=(jax.ShapeDtypeStruct((e_pad, 16), f32),
                   jax.ShapeDtypeStruct((e_pad, 128), f32)),
    )

    precompute = pl.pallas_call(
        _pre_body,
        grid=(gn_blocks,),
        in_specs=[_rows(BN, d_node), _full((d_node, 96)), _full((1, 96))],
        out_specs=(_rows(BN, 32), _rows(BN, 64)),
        out_shape=(jax.ShapeDtypeStruct((n_nodes, 32), f32),
                   jax.ShapeDtypeStruct((n_nodes, 64), f32)),
    )

    node_mlp = pl.pallas_call(
        _node_body,
        grid=(gn_blocks,),
        in_specs=[pl.BlockSpec((NC, BN, d_node), lambda i: (0, i, 0)),
                  _full((d_node, d_node)), _full((1, d_node)),
                  _full((1, d_node)), _full((1, d_node))],
        out_specs=_rows(BN, d_node),
        out_shape=jax.ShapeDtypeStruct((n_nodes, d_node), f32),
    )

    e = e0
    for _ in range(4):
        t1, t2 = precompute(x, wcat, bcat)
        s1, s2 = gather_k(t1, t2, rowp, colp)
        e, fm = edgeflow(s1, s2, e,
                         wc, r2d(ge1), r2d(bbe1),
                         we2, r2d(be2), r2d(ge2), r2d(bbe2),
                         vb, r2d(gf1), r2d(bbf1),
                         wf2, r2d(bf2), r2d(gf2), r2d(bbf2))
        partials = scatter_k(fm, rowe)
        x = node_mlp(partials, wn, r2d(bn), r2d(gn), r2d(bbn))

    return x, e[:n_edges]
